# baseline (speedup 1.0000x reference)
import jax
import jax.numpy as jnp
from jax import lax
from jax.experimental import pallas as pl
from jax.experimental.pallas import tpu as pltpu
from jax.experimental.pallas import tpu_sc as plsc

N = 10000
N_PAD = 10112
E = 320000
D = 128
H = 8
C1 = 16
NCORES = 2
NSUB = 16
NW = NCORES * NSUB
CHUNK = 128
NCH = 80
E_PAD = NCH * CHUNK * NW
ROWS_PER_SUB = N_PAD // NSUB
RBLK = 1264

_HI = lax.Precision.HIGHEST
_F32 = jnp.float32


def _dot(a, b):
    return jnp.dot(a, b, precision=_HI, preferred_element_type=_F32)


def _proj_body(x_ref, w_ref, ms_ref, md_ref, h_ref, s_ref, d_ref):
    h = _dot(x_ref[...], w_ref[...])
    h_ref[...] = h
    s_ref[...] = _dot(h, ms_ref[...])
    d_ref[...] = _dot(h, md_ref[...])


def _tc_project(x, w, ms, md):
    return pl.pallas_call(
        _proj_body,
        grid=(N_PAD // RBLK,),
        in_specs=[
            pl.BlockSpec((RBLK, D), lambda i: (i, 0)),
            pl.BlockSpec((D, D), lambda i: (0, 0)),
            pl.BlockSpec((D, 16), lambda i: (0, 0)),
            pl.BlockSpec((D, 16), lambda i: (0, 0)),
        ],
        out_specs=[
            pl.BlockSpec((RBLK, D), lambda i: (i, 0)),
            pl.BlockSpec((RBLK, 16), lambda i: (i, 0)),
            pl.BlockSpec((RBLK, 16), lambda i: (i, 0)),
        ],
        out_shape=[
            jax.ShapeDtypeStruct((N_PAD, D), _F32),
            jax.ShapeDtypeStruct((N_PAD, 16), _F32),
            jax.ShapeDtypeStruct((N_PAD, 16), _F32),
        ],
    )(x, w, ms, md)


def _mid_body(num_ref, den_ref, rep_ref, b1_ref, bnw_ref, bnb_ref,
              w2_ref, ms_ref, md_ref, h_ref, s_ref, d_ref):
    nm = num_ref[0] + num_ref[1]
    dn = den_ref[0] + den_ref[1]
    rep = _dot(dn, rep_ref[...])
    g = nm / (rep + 1e-16) + b1_ref[...]
    t = g * bnw_ref[...] + bnb_ref[...]
    t = jnp.where(t > 0.0, t, jnp.exp(t) - 1.0)
    h2 = _dot(t, w2_ref[...])
    h_ref[...] = h2
    s_ref[...] = _dot(h2, ms_ref[...])
    d_ref[...] = _dot(h2, md_ref[...])


def _tc_mid(num, den, rep, b1, bnw, bnb, w2, ms, md):
    return pl.pallas_call(
        _mid_body,
        grid=(N_PAD // RBLK,),
        in_specs=[
            pl.BlockSpec((NCORES, RBLK, D), lambda i: (0, i, 0)),
            pl.BlockSpec((NCORES, RBLK, 16), lambda i: (0, i, 0)),
            pl.BlockSpec((16, D), lambda i: (0, 0)),
            pl.BlockSpec((1, D), lambda i: (0, 0)),
            pl.BlockSpec((1, D), lambda i: (0, 0)),
            pl.BlockSpec((1, D), lambda i: (0, 0)),
            pl.BlockSpec((D, D), lambda i: (0, 0)),
            pl.BlockSpec((D, 16), lambda i: (0, 0)),
            pl.BlockSpec((D, 16), lambda i: (0, 0)),
        ],
        out_specs=[
            pl.BlockSpec((RBLK, D), lambda i: (i, 0)),
            pl.BlockSpec((RBLK, 16), lambda i: (i, 0)),
            pl.BlockSpec((RBLK, 16), lambda i: (i, 0)),
        ],
        out_shape=[
            jax.ShapeDtypeStruct((N_PAD, D), _F32),
            jax.ShapeDtypeStruct((N_PAD, 16), _F32),
            jax.ShapeDtypeStruct((N_PAD, 16), _F32),
        ],
    )(num, den, rep, b1, bnw, bnb, w2, ms, md)


def _fin_body(num_ref, den_ref, rep_ref, b2_ref, o_ref):
    nm = num_ref[0] + num_ref[1]
    dn = den_ref[0] + den_ref[1]
    rep = _dot(dn, rep_ref[...])
    o_ref[...] = nm / (rep + 1e-16) + b2_ref[...]


def _tc_final(num, den, rep, b2):
    return pl.pallas_call(
        _fin_body,
        grid=(N_PAD // RBLK,),
        in_specs=[
            pl.BlockSpec((NCORES, RBLK, D), lambda i: (0, i, 0)),
            pl.BlockSpec((NCORES, RBLK, 16), lambda i: (0, i, 0)),
            pl.BlockSpec((16, D), lambda i: (0, 0)),
            pl.BlockSpec((1, D), lambda i: (0, 0)),
        ],
        out_specs=pl.BlockSpec((RBLK, D), lambda i: (i, 0)),
        out_shape=jax.ShapeDtypeStruct((N_PAD, D), _F32),
    )(num, den, rep, b2)


def _sc_edge_body(src2_hbm, dst2_hbm, stab_hbm, dtab_hbm, h_hbm,
                  z128_hbm, z16_hbm,
                  num_hbm, den_hbm,
                  sidx0, didx0, sidx1, didx1,
                  hrows0, hrows1, srows, drows, wbuf,
                  num_sh, den_sh, semi, semg0, semg1, semgs, semz, semsc0, semsc1, semds):
    cid = lax.axis_index("c")
    sid = lax.axis_index("s")
    wid = sid * NCORES + cid

    IDX = ((sidx0, didx0), (sidx1, didx1))
    HROWS = (hrows0, hrows1)
    SEMG = (semg0, semg1)
    SEMSC = (semsc0, semsc1)

    def issue_idx(k, bi):
        pltpu.async_copy(src2_hbm.at[wid * NCH + k], IDX[bi][0], semi)
        pltpu.async_copy(dst2_hbm.at[wid * NCH + k], IDX[bi][1], semi)

    def wait_idx(bi):
        pltpu.make_async_copy(src2_hbm.at[0], IDX[bi][0], semi).wait()
        pltpu.make_async_copy(dst2_hbm.at[0], IDX[bi][1], semi).wait()

    def issue_h(bi):
        pltpu.async_copy(h_hbm.at[IDX[bi][0]], HROWS[bi], SEMG[bi])

    def wait_h(bi):
        pltpu.make_async_copy(h_hbm.at[IDX[bi][0]], HROWS[bi], SEMG[bi]).wait()

    def issue_sd(bi):
        pltpu.async_copy(stab_hbm.at[IDX[bi][0]], srows, semgs)
        pltpu.async_copy(dtab_hbm.at[IDX[bi][1]], drows, semgs)

    def wait_sd(bi):
        pltpu.make_async_copy(stab_hbm.at[IDX[bi][0]], srows, semgs).wait()
        pltpu.make_async_copy(dtab_hbm.at[IDX[bi][1]], drows, semgs).wait()

    r0 = sid * ROWS_PER_SUB
    pltpu.async_copy(z128_hbm.at[pl.ds(r0, ROWS_PER_SUB)],
                     num_sh.at[pl.ds(r0, ROWS_PER_SUB)], semz)
    pltpu.async_copy(z16_hbm.at[pl.ds(r0, ROWS_PER_SUB)],
                     den_sh.at[pl.ds(r0, ROWS_PER_SUB)], semz)
    issue_idx(0, 0)
    wait_idx(0)
    issue_h(0)
    issue_sd(0)
    issue_idx(1, 1)
    pltpu.make_async_copy(z128_hbm.at[pl.ds(r0, ROWS_PER_SUB)],
                          num_sh.at[pl.ds(r0, ROWS_PER_SUB)], semz).wait()
    pltpu.make_async_copy(z16_hbm.at[pl.ds(r0, ROWS_PER_SUB)],
                          den_sh.at[pl.ds(r0, ROWS_PER_SUB)], semz).wait()
    plsc.subcore_barrier()

    def outer(j, carry):
        for b in range(2):
            k = 2 * j + b
            nb = 1 - b
            hrows = HROWS[b]
            didx = IDX[b][1]

            @pl.when(k < NCH - 1)
            def _():
                wait_idx(nb)

                @pl.when(k >= 1)
                def _():
                    pltpu.make_async_copy(
                        HROWS[nb], num_sh.at[IDX[nb][1]], SEMSC[nb]).wait()

                issue_h(nb)

            wait_sd(b)

            @pl.when(k >= 1)
            def _():
                pltpu.make_async_copy(wbuf, den_sh.at[didx], semds).wait()

            def w_body(e, c):
                t = srows[e, :] + drows[e, :]
                t = jnp.where(t > 0.0, t, 0.2 * t)
                wbuf[e, :] = jnp.exp(t)
                return c

            lax.fori_loop(0, CHUNK, w_body, 0, unroll=4)
            wait_h(b)

            def m_body(e, c):
                wrow = wbuf[e, :]
                for hd in range(H):
                    hrows[e, pl.ds(hd * C1, C1)] = (
                        hrows[e, pl.ds(hd * C1, C1)] * wrow[hd])
                return c

            lax.fori_loop(0, CHUNK, m_body, 0, unroll=4)
            pltpu.async_copy(hrows, num_sh.at[didx], SEMSC[b], add=True)
            pltpu.async_copy(wbuf, den_sh.at[didx], semds, add=True)

            @pl.when(k < NCH - 2)
            def _():
                issue_idx(k + 2, b)

            @pl.when(k < NCH - 1)
            def _():
                issue_sd(nb)
        return carry

    lax.fori_loop(0, NCH // 2, outer, 0)
    pltpu.make_async_copy(HROWS[0], num_sh.at[IDX[0][1]], SEMSC[0]).wait()
    pltpu.make_async_copy(HROWS[1], num_sh.at[IDX[1][1]], SEMSC[1]).wait()
    pltpu.make_async_copy(wbuf, den_sh.at[IDX[1][1]], semds).wait()
    plsc.subcore_barrier()

    pltpu.sync_copy(num_sh.at[pl.ds(r0, ROWS_PER_SUB)],
                    num_hbm.at[cid, pl.ds(r0, ROWS_PER_SUB)])
    pltpu.sync_copy(den_sh.at[pl.ds(r0, ROWS_PER_SUB)],
                    den_hbm.at[cid, pl.ds(r0, ROWS_PER_SUB)])


def _make_sc_edge():
  return pl.kernel(
    _sc_edge_body,
    out_type=[
        jax.ShapeDtypeStruct((NCORES, N_PAD, D), _F32),
        jax.ShapeDtypeStruct((NCORES, N_PAD, 16), _F32),
    ],
    mesh=plsc.VectorSubcoreMesh(core_axis_name="c", subcore_axis_name="s",
                                num_cores=NCORES, num_subcores=NSUB),
    compiler_params=pltpu.CompilerParams(use_tc_tiling_on_sc=False),
    scratch_types=[
        pltpu.VMEM((CHUNK,), jnp.int32),
        pltpu.VMEM((CHUNK,), jnp.int32),
        pltpu.VMEM((CHUNK,), jnp.int32),
        pltpu.VMEM((CHUNK,), jnp.int32),
        pltpu.VMEM((CHUNK, D), _F32),
        pltpu.VMEM((CHUNK, D), _F32),
        pltpu.VMEM((CHUNK, 16), _F32),
        pltpu.VMEM((CHUNK, 16), _F32),
        pltpu.VMEM((CHUNK, 16), _F32),
        pltpu.VMEM_SHARED((N_PAD, D), _F32),
        pltpu.VMEM_SHARED((N_PAD, 16), _F32),
        pltpu.SemaphoreType.DMA,
        pltpu.SemaphoreType.DMA,
        pltpu.SemaphoreType.DMA,
        pltpu.SemaphoreType.DMA,
        pltpu.SemaphoreType.DMA,
        pltpu.SemaphoreType.DMA,
        pltpu.SemaphoreType.DMA,
        pltpu.SemaphoreType.DMA,
    ],
  )


def kernel(x, edge_index, W1, a_src1, a_dst1, b1, bn_w, bn_b,
           W2, a_src2, a_dst2, b2):
    src = edge_index[0]
    dst = edge_index[1]

    eye = jnp.eye(H, dtype=_F32)
    a1s = a_src1.reshape(H, C1)
    a1d = a_dst1.reshape(H, C1)
    ms1h = (a1s[:, :, None] * eye[:, None, :]).reshape(H * C1, H)
    md1h = (a1d[:, :, None] * eye[:, None, :]).reshape(H * C1, H)
    ms1 = jnp.concatenate([ms1h, ms1h], axis=1)
    md1 = jnp.concatenate([md1h, md1h], axis=1)
    ms2 = jnp.tile(a_src2.reshape(D, 1), (1, 16))
    md2 = jnp.tile(a_dst2.reshape(D, 1), (1, 16))
    rep1 = jnp.concatenate(
        [jnp.kron(eye, jnp.ones((1, C1), _F32)), jnp.zeros((H, D), _F32)], axis=0)
    rep2 = jnp.full((16, D), 1.0 / 16.0, _F32)

    bnw_s = (bn_w / jnp.sqrt(1.0 + 1e-5)).reshape(1, D)
    z128 = jnp.zeros((N_PAD, D), _F32)
    z16 = jnp.zeros((N_PAD, 16), _F32)

    sc_edge = _make_sc_edge()
    pad = N + (jnp.arange(E_PAD - E, dtype=jnp.int32) % (N_PAD - N))
    src2 = jnp.concatenate([src, pad]).reshape(NCH * NW, CHUNK)
    dst2 = jnp.concatenate([dst, pad]).reshape(NCH * NW, CHUNK)
    xp = jnp.pad(x, ((0, N_PAD - N), (0, 0)))
    h1, s1, d1 = _tc_project(xp, W1, ms1, md1)
    num1, den1 = sc_edge(src2, dst2, s1, d1, h1, z128, z16)
    h2, s2, d2 = _tc_mid(num1, den1, rep1, b1.reshape(1, D), bnw_s,
                         bn_b.reshape(1, D), W2, ms2, md2)
    num2, den2 = sc_edge(src2, dst2, s2, d2, h2, z128, z16)
    return _tc_final(num2, den2, rep2, b2.reshape(1, D))[:N]


# speedup vs baseline: 1.0030x; 1.0030x over previous
import jax
import jax.numpy as jnp
from jax import lax
from jax.experimental import pallas as pl
from jax.experimental.pallas import tpu as pltpu
from jax.experimental.pallas import tpu_sc as plsc

N = 10000
N_PAD = 10112
E = 320000
D = 128
H = 8
C1 = 16
NCORES = 2
NSUB = 16
NW = NCORES * NSUB
CHUNK = 128
NCH = 80
E_PAD = NCH * CHUNK * NW
ROWS_PER_SUB = N_PAD // NSUB
RBLK = 1264

_HI = lax.Precision.HIGHEST
_F32 = jnp.float32


def _dot(a, b):
    return jnp.dot(a, b, precision=_HI, preferred_element_type=_F32)


def _proj_body(x_ref, w_ref, ms_ref, md_ref, h_ref, s_ref, d_ref):
    h = _dot(x_ref[...], w_ref[...])
    h_ref[...] = h
    s_ref[...] = _dot(h, ms_ref[...])
    d_ref[...] = _dot(h, md_ref[...])


def _tc_project(x, w, ms, md):
    return pl.pallas_call(
        _proj_body,
        grid=(N_PAD // RBLK,),
        in_specs=[
            pl.BlockSpec((RBLK, D), lambda i: (i, 0)),
            pl.BlockSpec((D, D), lambda i: (0, 0)),
            pl.BlockSpec((D, 16), lambda i: (0, 0)),
            pl.BlockSpec((D, 16), lambda i: (0, 0)),
        ],
        out_specs=[
            pl.BlockSpec((RBLK, D), lambda i: (i, 0)),
            pl.BlockSpec((RBLK, 16), lambda i: (i, 0)),
            pl.BlockSpec((RBLK, 16), lambda i: (i, 0)),
        ],
        out_shape=[
            jax.ShapeDtypeStruct((N_PAD, D), _F32),
            jax.ShapeDtypeStruct((N_PAD, 16), _F32),
            jax.ShapeDtypeStruct((N_PAD, 16), _F32),
        ],
    )(x, w, ms, md)


def _mid_body(num_ref, den_ref, rep_ref, b1_ref, bnw_ref, bnb_ref,
              w2_ref, ms_ref, md_ref, h_ref, s_ref, d_ref):
    nm = num_ref[0] + num_ref[1]
    dn = den_ref[0] + den_ref[1]
    rep = _dot(dn, rep_ref[...])
    g = nm / (rep + 1e-16) + b1_ref[...]
    t = g * bnw_ref[...] + bnb_ref[...]
    t = jnp.where(t > 0.0, t, jnp.exp(t) - 1.0)
    h2 = _dot(t, w2_ref[...])
    h_ref[...] = h2
    s_ref[...] = _dot(h2, ms_ref[...])
    d_ref[...] = _dot(h2, md_ref[...])


def _tc_mid(num, den, rep, b1, bnw, bnb, w2, ms, md):
    return pl.pallas_call(
        _mid_body,
        grid=(N_PAD // RBLK,),
        in_specs=[
            pl.BlockSpec((NCORES, RBLK, D), lambda i: (0, i, 0)),
            pl.BlockSpec((NCORES, RBLK, 16), lambda i: (0, i, 0)),
            pl.BlockSpec((16, D), lambda i: (0, 0)),
            pl.BlockSpec((1, D), lambda i: (0, 0)),
            pl.BlockSpec((1, D), lambda i: (0, 0)),
            pl.BlockSpec((1, D), lambda i: (0, 0)),
            pl.BlockSpec((D, D), lambda i: (0, 0)),
            pl.BlockSpec((D, 16), lambda i: (0, 0)),
            pl.BlockSpec((D, 16), lambda i: (0, 0)),
        ],
        out_specs=[
            pl.BlockSpec((RBLK, D), lambda i: (i, 0)),
            pl.BlockSpec((RBLK, 16), lambda i: (i, 0)),
            pl.BlockSpec((RBLK, 16), lambda i: (i, 0)),
        ],
        out_shape=[
            jax.ShapeDtypeStruct((N_PAD, D), _F32),
            jax.ShapeDtypeStruct((N_PAD, 16), _F32),
            jax.ShapeDtypeStruct((N_PAD, 16), _F32),
        ],
    )(num, den, rep, b1, bnw, bnb, w2, ms, md)


def _fin_body(num_ref, den_ref, rep_ref, b2_ref, o_ref):
    nm = num_ref[0] + num_ref[1]
    dn = den_ref[0] + den_ref[1]
    rep = _dot(dn, rep_ref[...])
    o_ref[...] = nm / (rep + 1e-16) + b2_ref[...]


def _tc_final(num, den, rep, b2):
    return pl.pallas_call(
        _fin_body,
        grid=(N_PAD // RBLK,),
        in_specs=[
            pl.BlockSpec((NCORES, RBLK, D), lambda i: (0, i, 0)),
            pl.BlockSpec((NCORES, RBLK, 16), lambda i: (0, i, 0)),
            pl.BlockSpec((16, D), lambda i: (0, 0)),
            pl.BlockSpec((1, D), lambda i: (0, 0)),
        ],
        out_specs=pl.BlockSpec((RBLK, D), lambda i: (i, 0)),
        out_shape=jax.ShapeDtypeStruct((N_PAD, D), _F32),
    )(num, den, rep, b2)


def _sc_edge_body(src2_hbm, dst2_hbm, stab_hbm, dtab_hbm, h_hbm,
                  z128_hbm, z16_hbm,
                  num_hbm, den_hbm,
                  sidx0, didx0, sidx1, didx1,
                  hrows0, hrows1, srows, drows, wbuf,
                  num_sh, den_sh, semi, semg0, semg1, semgs, semz, semsc0, semsc1, semds):
    cid = lax.axis_index("c")
    sid = lax.axis_index("s")
    wid = sid * NCORES + cid

    IDX = ((sidx0, didx0), (sidx1, didx1))
    HROWS = (hrows0, hrows1)
    SEMG = (semg0, semg1)
    SEMSC = (semsc0, semsc1)

    def issue_idx(k, bi):
        pltpu.async_copy(src2_hbm.at[wid * NCH + k], IDX[bi][0], semi)
        pltpu.async_copy(dst2_hbm.at[wid * NCH + k], IDX[bi][1], semi)

    def wait_idx(bi):
        pltpu.make_async_copy(src2_hbm.at[0], IDX[bi][0], semi).wait()
        pltpu.make_async_copy(dst2_hbm.at[0], IDX[bi][1], semi).wait()

    def issue_h(bi):
        pltpu.async_copy(h_hbm.at[IDX[bi][0]], HROWS[bi], SEMG[bi])

    def wait_h(bi):
        pltpu.make_async_copy(h_hbm.at[IDX[bi][0]], HROWS[bi], SEMG[bi]).wait()

    def issue_sd(bi):
        pltpu.async_copy(stab_hbm.at[IDX[bi][0]], srows, semgs)
        pltpu.async_copy(dtab_hbm.at[IDX[bi][1]], drows, semgs)

    def wait_sd(bi):
        pltpu.make_async_copy(stab_hbm.at[IDX[bi][0]], srows, semgs).wait()
        pltpu.make_async_copy(dtab_hbm.at[IDX[bi][1]], drows, semgs).wait()

    r0 = sid * ROWS_PER_SUB
    pltpu.async_copy(z128_hbm.at[pl.ds(r0, ROWS_PER_SUB)],
                     num_sh.at[pl.ds(r0, ROWS_PER_SUB)], semz)
    pltpu.async_copy(z16_hbm.at[pl.ds(r0, ROWS_PER_SUB)],
                     den_sh.at[pl.ds(r0, ROWS_PER_SUB)], semz)
    issue_idx(0, 0)
    wait_idx(0)
    issue_h(0)
    issue_sd(0)
    issue_idx(1, 1)
    pltpu.make_async_copy(z128_hbm.at[pl.ds(r0, ROWS_PER_SUB)],
                          num_sh.at[pl.ds(r0, ROWS_PER_SUB)], semz).wait()
    pltpu.make_async_copy(z16_hbm.at[pl.ds(r0, ROWS_PER_SUB)],
                          den_sh.at[pl.ds(r0, ROWS_PER_SUB)], semz).wait()
    plsc.subcore_barrier()

    def outer(j, carry):
        for b in range(2):
            k = 2 * j + b
            nb = 1 - b
            hrows = HROWS[b]
            didx = IDX[b][1]

            @pl.when(k < NCH - 1)
            def _():
                wait_idx(nb)

                @pl.when(k >= 1)
                def _():
                    pltpu.make_async_copy(
                        HROWS[nb], num_sh.at[IDX[nb][1]], SEMSC[nb]).wait()

                issue_h(nb)

            wait_sd(b)

            @pl.when(k >= 1)
            def _():
                pltpu.make_async_copy(wbuf, den_sh.at[didx], semds).wait()

            def w_body(e, c):
                t = srows[e, :] + drows[e, :]
                t = jnp.where(t > 0.0, t, 0.2 * t)
                wbuf[e, :] = jnp.exp(t)
                return c

            lax.fori_loop(0, CHUNK, w_body, 0, unroll=2)
            wait_h(b)

            def m_body(e, c):
                wrow = wbuf[e, :]
                for hd in range(H):
                    hrows[e, pl.ds(hd * C1, C1)] = (
                        hrows[e, pl.ds(hd * C1, C1)] * wrow[hd])
                return c

            lax.fori_loop(0, CHUNK, m_body, 0, unroll=2)
            pltpu.async_copy(hrows, num_sh.at[didx], SEMSC[b], add=True)
            pltpu.async_copy(wbuf, den_sh.at[didx], semds, add=True)

            @pl.when(k < NCH - 2)
            def _():
                issue_idx(k + 2, b)

            @pl.when(k < NCH - 1)
            def _():
                issue_sd(nb)
        return carry

    lax.fori_loop(0, NCH // 2, outer, 0)
    pltpu.make_async_copy(HROWS[0], num_sh.at[IDX[0][1]], SEMSC[0]).wait()
    pltpu.make_async_copy(HROWS[1], num_sh.at[IDX[1][1]], SEMSC[1]).wait()
    pltpu.make_async_copy(wbuf, den_sh.at[IDX[1][1]], semds).wait()
    plsc.subcore_barrier()

    pltpu.sync_copy(num_sh.at[pl.ds(r0, ROWS_PER_SUB)],
                    num_hbm.at[cid, pl.ds(r0, ROWS_PER_SUB)])
    pltpu.sync_copy(den_sh.at[pl.ds(r0, ROWS_PER_SUB)],
                    den_hbm.at[cid, pl.ds(r0, ROWS_PER_SUB)])


def _make_sc_edge():
  return pl.kernel(
    _sc_edge_body,
    out_type=[
        jax.ShapeDtypeStruct((NCORES, N_PAD, D), _F32),
        jax.ShapeDtypeStruct((NCORES, N_PAD, 16), _F32),
    ],
    mesh=plsc.VectorSubcoreMesh(core_axis_name="c", subcore_axis_name="s",
                                num_cores=NCORES, num_subcores=NSUB),
    compiler_params=pltpu.CompilerParams(use_tc_tiling_on_sc=False),
    scratch_types=[
        pltpu.VMEM((CHUNK,), jnp.int32),
        pltpu.VMEM((CHUNK,), jnp.int32),
        pltpu.VMEM((CHUNK,), jnp.int32),
        pltpu.VMEM((CHUNK,), jnp.int32),
        pltpu.VMEM((CHUNK, D), _F32),
        pltpu.VMEM((CHUNK, D), _F32),
        pltpu.VMEM((CHUNK, 16), _F32),
        pltpu.VMEM((CHUNK, 16), _F32),
        pltpu.VMEM((CHUNK, 16), _F32),
        pltpu.VMEM_SHARED((N_PAD, D), _F32),
        pltpu.VMEM_SHARED((N_PAD, 16), _F32),
        pltpu.SemaphoreType.DMA,
        pltpu.SemaphoreType.DMA,
        pltpu.SemaphoreType.DMA,
        pltpu.SemaphoreType.DMA,
        pltpu.SemaphoreType.DMA,
        pltpu.SemaphoreType.DMA,
        pltpu.SemaphoreType.DMA,
        pltpu.SemaphoreType.DMA,
    ],
  )


def kernel(x, edge_index, W1, a_src1, a_dst1, b1, bn_w, bn_b,
           W2, a_src2, a_dst2, b2):
    src = edge_index[0]
    dst = edge_index[1]

    eye = jnp.eye(H, dtype=_F32)
    a1s = a_src1.reshape(H, C1)
    a1d = a_dst1.reshape(H, C1)
    ms1h = (a1s[:, :, None] * eye[:, None, :]).reshape(H * C1, H)
    md1h = (a1d[:, :, None] * eye[:, None, :]).reshape(H * C1, H)
    ms1 = jnp.concatenate([ms1h, ms1h], axis=1)
    md1 = jnp.concatenate([md1h, md1h], axis=1)
    ms2 = jnp.tile(a_src2.reshape(D, 1), (1, 16))
    md2 = jnp.tile(a_dst2.reshape(D, 1), (1, 16))
    rep1 = jnp.concatenate(
        [jnp.kron(eye, jnp.ones((1, C1), _F32)), jnp.zeros((H, D), _F32)], axis=0)
    rep2 = jnp.full((16, D), 1.0 / 16.0, _F32)

    bnw_s = (bn_w / jnp.sqrt(1.0 + 1e-5)).reshape(1, D)
    z128 = jnp.zeros((N_PAD, D), _F32)
    z16 = jnp.zeros((N_PAD, 16), _F32)

    sc_edge = _make_sc_edge()
    pad = N + (jnp.arange(E_PAD - E, dtype=jnp.int32) % (N_PAD - N))
    src2 = jnp.concatenate([src, pad]).reshape(NCH * NW, CHUNK)
    dst2 = jnp.concatenate([dst, pad]).reshape(NCH * NW, CHUNK)
    xp = jnp.pad(x, ((0, N_PAD - N), (0, 0)))
    h1, s1, d1 = _tc_project(xp, W1, ms1, md1)
    num1, den1 = sc_edge(src2, dst2, s1, d1, h1, z128, z16)
    h2, s2, d2 = _tc_mid(num1, den1, rep1, b1.reshape(1, D), bnw_s,
                         bn_b.reshape(1, D), W2, ms2, md2)
    num2, den2 = sc_edge(src2, dst2, s2, d2, h2, z128, z16)
    return _tc_final(num2, den2, rep2, b2.reshape(1, D))[:N]


# speedup vs baseline: 1.3480x; 1.3440x over previous
import jax
import jax.numpy as jnp
from jax import lax
from jax.experimental import pallas as pl
from jax.experimental.pallas import tpu as pltpu
from jax.experimental.pallas import tpu_sc as plsc

N = 10000
N_PAD = 10112
E = 320000
D = 128
H = 8
C1 = 16
NCORES = 2
NSUB = 16
NW = NCORES * NSUB
CHUNK = 128
NCH = 80
E_PAD = NCH * CHUNK * NW
ROWS_PER_SUB = N_PAD // NSUB
RBLK = 1264

_HI = lax.Precision.HIGHEST
_F32 = jnp.float32


def _dot(a, b):
    return jnp.dot(a, b, precision=_HI, preferred_element_type=_F32)


def _proj_body(x_ref, w_ref, ms_ref, md_ref, h_ref, s_ref, d_ref):
    h = _dot(x_ref[...], w_ref[...])
    h_ref[...] = h
    s_ref[...] = _dot(h, ms_ref[...])
    d_ref[...] = _dot(h, md_ref[...])


def _tc_project(x, w, ms, md):
    return pl.pallas_call(
        _proj_body,
        grid=(N_PAD // RBLK,),
        in_specs=[
            pl.BlockSpec((RBLK, D), lambda i: (i, 0)),
            pl.BlockSpec((D, D), lambda i: (0, 0)),
            pl.BlockSpec((D, 16), lambda i: (0, 0)),
            pl.BlockSpec((D, 16), lambda i: (0, 0)),
        ],
        out_specs=[
            pl.BlockSpec((RBLK, D), lambda i: (i, 0)),
            pl.BlockSpec((RBLK, 16), lambda i: (i, 0)),
            pl.BlockSpec((RBLK, 16), lambda i: (i, 0)),
        ],
        out_shape=[
            jax.ShapeDtypeStruct((N_PAD, D), _F32),
            jax.ShapeDtypeStruct((N_PAD, 16), _F32),
            jax.ShapeDtypeStruct((N_PAD, 16), _F32),
        ],
    )(x, w, ms, md)


def _mid_body(num_ref, den_ref, rep_ref, b1_ref, bnw_ref, bnb_ref,
              w2_ref, ms_ref, md_ref, h_ref, s_ref, d_ref):
    nm = num_ref[0] + num_ref[1]
    dn = den_ref[0] + den_ref[1]
    rep = _dot(dn, rep_ref[...])
    g = nm / (rep + 1e-16) + b1_ref[...]
    t = g * bnw_ref[...] + bnb_ref[...]
    t = jnp.where(t > 0.0, t, jnp.exp(t) - 1.0)
    h2 = _dot(t, w2_ref[...])
    h_ref[...] = h2
    s_ref[...] = _dot(h2, ms_ref[...])
    d_ref[...] = _dot(h2, md_ref[...])


def _tc_mid(num, den, rep, b1, bnw, bnb, w2, ms, md):
    return pl.pallas_call(
        _mid_body,
        grid=(N_PAD // RBLK,),
        in_specs=[
            pl.BlockSpec((NCORES, RBLK, D), lambda i: (0, i, 0)),
            pl.BlockSpec((NCORES, RBLK, 16), lambda i: (0, i, 0)),
            pl.BlockSpec((16, D), lambda i: (0, 0)),
            pl.BlockSpec((1, D), lambda i: (0, 0)),
            pl.BlockSpec((1, D), lambda i: (0, 0)),
            pl.BlockSpec((1, D), lambda i: (0, 0)),
            pl.BlockSpec((D, D), lambda i: (0, 0)),
            pl.BlockSpec((D, 16), lambda i: (0, 0)),
            pl.BlockSpec((D, 16), lambda i: (0, 0)),
        ],
        out_specs=[
            pl.BlockSpec((RBLK, D), lambda i: (i, 0)),
            pl.BlockSpec((RBLK, 16), lambda i: (i, 0)),
            pl.BlockSpec((RBLK, 16), lambda i: (i, 0)),
        ],
        out_shape=[
            jax.ShapeDtypeStruct((N_PAD, D), _F32),
            jax.ShapeDtypeStruct((N_PAD, 16), _F32),
            jax.ShapeDtypeStruct((N_PAD, 16), _F32),
        ],
    )(num, den, rep, b1, bnw, bnb, w2, ms, md)


def _fin_body(num_ref, den_ref, rep_ref, b2_ref, o_ref):
    nm = num_ref[0] + num_ref[1]
    dn = den_ref[0] + den_ref[1]
    rep = _dot(dn, rep_ref[...])
    o_ref[...] = nm / (rep + 1e-16) + b2_ref[...]


def _tc_final(num, den, rep, b2):
    return pl.pallas_call(
        _fin_body,
        grid=(N_PAD // RBLK,),
        in_specs=[
            pl.BlockSpec((NCORES, RBLK, D), lambda i: (0, i, 0)),
            pl.BlockSpec((NCORES, RBLK, 16), lambda i: (0, i, 0)),
            pl.BlockSpec((16, D), lambda i: (0, 0)),
            pl.BlockSpec((1, D), lambda i: (0, 0)),
        ],
        out_specs=pl.BlockSpec((RBLK, D), lambda i: (i, 0)),
        out_shape=jax.ShapeDtypeStruct((N_PAD, D), _F32),
    )(num, den, rep, b2)


def _sc_edge_body(src2_hbm, dst2_hbm, stab_hbm, dtab_hbm, h_hbm,
                  z128_hbm, z16_hbm,
                  num_hbm, den_hbm,
                  sidx0, didx0, sidx1, didx1,
                  hrows0, hrows1, srows, drows, wbuf,
                  num_sh, den_sh, semi, semg0, semg1, semgs, semz, semsc0, semsc1, semds):
    cid = lax.axis_index("c")
    sid = lax.axis_index("s")
    wid = sid * NCORES + cid

    IDX = ((sidx0, didx0), (sidx1, didx1))
    HROWS = (hrows0, hrows1)
    SEMG = (semg0, semg1)
    SEMSC = (semsc0, semsc1)

    def issue_idx(k, bi):
        pltpu.async_copy(src2_hbm.at[wid * NCH + k], IDX[bi][0], semi)
        pltpu.async_copy(dst2_hbm.at[wid * NCH + k], IDX[bi][1], semi)

    def wait_idx(bi):
        pltpu.make_async_copy(src2_hbm.at[0], IDX[bi][0], semi).wait()
        pltpu.make_async_copy(dst2_hbm.at[0], IDX[bi][1], semi).wait()

    def issue_h(bi):
        pltpu.async_copy(h_hbm.at[IDX[bi][0]], HROWS[bi], SEMG[bi])

    def wait_h(bi):
        pltpu.make_async_copy(h_hbm.at[IDX[bi][0]], HROWS[bi], SEMG[bi]).wait()

    def issue_sd(bi):
        pltpu.async_copy(stab_hbm.at[IDX[bi][0]], srows, semgs)
        pltpu.async_copy(dtab_hbm.at[IDX[bi][1]], drows, semgs)

    def wait_sd(bi):
        pltpu.make_async_copy(stab_hbm.at[IDX[bi][0]], srows, semgs).wait()
        pltpu.make_async_copy(dtab_hbm.at[IDX[bi][1]], drows, semgs).wait()

    r0 = sid * ROWS_PER_SUB
    pltpu.async_copy(z128_hbm.at[pl.ds(r0, ROWS_PER_SUB)],
                     num_sh.at[pl.ds(r0, ROWS_PER_SUB)], semz)
    pltpu.async_copy(z16_hbm.at[pl.ds(r0, ROWS_PER_SUB)],
                     den_sh.at[pl.ds(r0, ROWS_PER_SUB)], semz)
    issue_idx(0, 0)
    wait_idx(0)
    issue_h(0)
    issue_sd(0)
    issue_idx(1, 1)
    pltpu.make_async_copy(z128_hbm.at[pl.ds(r0, ROWS_PER_SUB)],
                          num_sh.at[pl.ds(r0, ROWS_PER_SUB)], semz).wait()
    pltpu.make_async_copy(z16_hbm.at[pl.ds(r0, ROWS_PER_SUB)],
                          den_sh.at[pl.ds(r0, ROWS_PER_SUB)], semz).wait()
    plsc.subcore_barrier()

    def outer(j, carry):
        for b in range(2):
            k = 2 * j + b
            nb = 1 - b
            hrows = HROWS[b]
            didx = IDX[b][1]

            @pl.when(k < NCH - 1)
            def _():
                wait_idx(nb)

                @pl.when(k >= 1)
                def _():
                    pltpu.make_async_copy(
                        HROWS[nb], num_sh.at[IDX[nb][1]], SEMSC[nb]).wait()

                issue_h(nb)

            wait_sd(b)

            @pl.when(k >= 1)
            def _():
                pltpu.make_async_copy(wbuf, den_sh.at[didx], semds).wait()

            def w_body(e, c):
                t = srows[e, :] + drows[e, :]
                t = jnp.where(t > 0.0, t, 0.2 * t)
                wbuf[e, :] = jnp.exp(t)
                return c

            lax.fori_loop(0, CHUNK, w_body, 0)
            wait_h(b)

            def m_body(e, c):
                wrow = wbuf[e, :]
                for hd in range(H):
                    hrows[e, pl.ds(hd * C1, C1)] = (
                        hrows[e, pl.ds(hd * C1, C1)] * wrow[hd])
                return c

            lax.fori_loop(0, CHUNK, m_body, 0, unroll=2)
            pltpu.async_copy(hrows, num_sh.at[didx], SEMSC[b], add=True)
            pltpu.async_copy(wbuf, den_sh.at[didx], semds, add=True)

            @pl.when(k < NCH - 2)
            def _():
                issue_idx(k + 2, b)

            @pl.when(k < NCH - 1)
            def _():
                issue_sd(nb)
        return carry

    lax.fori_loop(0, NCH // 2, outer, 0)
    pltpu.make_async_copy(HROWS[0], num_sh.at[IDX[0][1]], SEMSC[0]).wait()
    pltpu.make_async_copy(HROWS[1], num_sh.at[IDX[1][1]], SEMSC[1]).wait()
    pltpu.make_async_copy(wbuf, den_sh.at[IDX[1][1]], semds).wait()
    plsc.subcore_barrier()

    pltpu.sync_copy(num_sh.at[pl.ds(r0, ROWS_PER_SUB)],
                    num_hbm.at[cid, pl.ds(r0, ROWS_PER_SUB)])
    pltpu.sync_copy(den_sh.at[pl.ds(r0, ROWS_PER_SUB)],
                    den_hbm.at[cid, pl.ds(r0, ROWS_PER_SUB)])


def _make_sc_edge():
  return pl.kernel(
    _sc_edge_body,
    out_type=[
        jax.ShapeDtypeStruct((NCORES, N_PAD, D), _F32),
        jax.ShapeDtypeStruct((NCORES, N_PAD, 16), _F32),
    ],
    mesh=plsc.VectorSubcoreMesh(core_axis_name="c", subcore_axis_name="s",
                                num_cores=NCORES, num_subcores=NSUB),
    compiler_params=pltpu.CompilerParams(use_tc_tiling_on_sc=False),
    scratch_types=[
        pltpu.VMEM((CHUNK,), jnp.int32),
        pltpu.VMEM((CHUNK,), jnp.int32),
        pltpu.VMEM((CHUNK,), jnp.int32),
        pltpu.VMEM((CHUNK,), jnp.int32),
        pltpu.VMEM((CHUNK, D), _F32),
        pltpu.VMEM((CHUNK, D), _F32),
        pltpu.VMEM((CHUNK, 16), _F32),
        pltpu.VMEM((CHUNK, 16), _F32),
        pltpu.VMEM((CHUNK, 16), _F32),
        pltpu.VMEM_SHARED((N_PAD, D), _F32),
        pltpu.VMEM_SHARED((N_PAD, 16), _F32),
        pltpu.SemaphoreType.DMA,
        pltpu.SemaphoreType.DMA,
        pltpu.SemaphoreType.DMA,
        pltpu.SemaphoreType.DMA,
        pltpu.SemaphoreType.DMA,
        pltpu.SemaphoreType.DMA,
        pltpu.SemaphoreType.DMA,
        pltpu.SemaphoreType.DMA,
    ],
  )


def kernel(x, edge_index, W1, a_src1, a_dst1, b1, bn_w, bn_b,
           W2, a_src2, a_dst2, b2):
    src = edge_index[0]
    dst = edge_index[1]

    eye = jnp.eye(H, dtype=_F32)
    a1s = a_src1.reshape(H, C1)
    a1d = a_dst1.reshape(H, C1)
    ms1h = (a1s[:, :, None] * eye[:, None, :]).reshape(H * C1, H)
    md1h = (a1d[:, :, None] * eye[:, None, :]).reshape(H * C1, H)
    ms1 = jnp.concatenate([ms1h, ms1h], axis=1)
    md1 = jnp.concatenate([md1h, md1h], axis=1)
    ms2 = jnp.tile(a_src2.reshape(D, 1), (1, 16))
    md2 = jnp.tile(a_dst2.reshape(D, 1), (1, 16))
    rep1 = jnp.concatenate(
        [jnp.kron(eye, jnp.ones((1, C1), _F32)), jnp.zeros((H, D), _F32)], axis=0)
    rep2 = jnp.full((16, D), 1.0 / 16.0, _F32)

    bnw_s = (bn_w / jnp.sqrt(1.0 + 1e-5)).reshape(1, D)
    z128 = jnp.zeros((N_PAD, D), _F32)
    z16 = jnp.zeros((N_PAD, 16), _F32)

    sc_edge = _make_sc_edge()
    pad = N + (jnp.arange(E_PAD - E, dtype=jnp.int32) % (N_PAD - N))
    src2 = jnp.concatenate([src, pad]).reshape(NCH * NW, CHUNK)
    dst2 = jnp.concatenate([dst, pad]).reshape(NCH * NW, CHUNK)
    xp = jnp.pad(x, ((0, N_PAD - N), (0, 0)))
    h1, s1, d1 = _tc_project(xp, W1, ms1, md1)
    num1, den1 = sc_edge(src2, dst2, s1, d1, h1, z128, z16)
    h2, s2, d2 = _tc_mid(num1, den1, rep1, b1.reshape(1, D), bnw_s,
                         bn_b.reshape(1, D), W2, ms2, md2)
    num2, den2 = sc_edge(src2, dst2, s2, d2, h2, z128, z16)
    return _tc_final(num2, den2, rep2, b2.reshape(1, D))[:N]


# speedup vs baseline: 1.5815x; 1.1732x over previous
import jax
import jax.numpy as jnp
from jax import lax
from jax.experimental import pallas as pl
from jax.experimental.pallas import tpu as pltpu
from jax.experimental.pallas import tpu_sc as plsc

N = 10000
N_PAD = 10112
E = 320000
D = 128
H = 8
C1 = 16
NCORES = 2
NSUB = 16
NW = NCORES * NSUB
CHUNK = 128
NCH = 80
E_PAD = NCH * CHUNK * NW
ROWS_PER_SUB = N_PAD // NSUB
RBLK = 1264

_HI = lax.Precision.HIGHEST
_F32 = jnp.float32


def _dot(a, b):
    return jnp.dot(a, b, precision=_HI, preferred_element_type=_F32)


def _proj_body(x_ref, w_ref, ms_ref, md_ref, h_ref, s_ref, d_ref):
    h = _dot(x_ref[...], w_ref[...])
    h_ref[...] = h
    s_ref[...] = _dot(h, ms_ref[...])
    d_ref[...] = _dot(h, md_ref[...])


def _tc_project(x, w, ms, md):
    return pl.pallas_call(
        _proj_body,
        grid=(N_PAD // RBLK,),
        in_specs=[
            pl.BlockSpec((RBLK, D), lambda i: (i, 0)),
            pl.BlockSpec((D, D), lambda i: (0, 0)),
            pl.BlockSpec((D, 16), lambda i: (0, 0)),
            pl.BlockSpec((D, 16), lambda i: (0, 0)),
        ],
        out_specs=[
            pl.BlockSpec((RBLK, D), lambda i: (i, 0)),
            pl.BlockSpec((RBLK, 16), lambda i: (i, 0)),
            pl.BlockSpec((RBLK, 16), lambda i: (i, 0)),
        ],
        out_shape=[
            jax.ShapeDtypeStruct((N_PAD, D), _F32),
            jax.ShapeDtypeStruct((N_PAD, 16), _F32),
            jax.ShapeDtypeStruct((N_PAD, 16), _F32),
        ],
    )(x, w, ms, md)


def _mid_body(num_ref, den_ref, rep_ref, b1_ref, bnw_ref, bnb_ref,
              w2_ref, ms_ref, md_ref, h_ref, s_ref, d_ref):
    nm = num_ref[0] + num_ref[1]
    dn = den_ref[0] + den_ref[1]
    rep = _dot(dn, rep_ref[...])
    g = nm / (rep + 1e-16) + b1_ref[...]
    t = g * bnw_ref[...] + bnb_ref[...]
    t = jnp.where(t > 0.0, t, jnp.exp(t) - 1.0)
    h2 = _dot(t, w2_ref[...])
    h_ref[...] = h2
    s_ref[...] = _dot(h2, ms_ref[...])
    d_ref[...] = _dot(h2, md_ref[...])


def _tc_mid(num, den, rep, b1, bnw, bnb, w2, ms, md):
    return pl.pallas_call(
        _mid_body,
        grid=(N_PAD // RBLK,),
        in_specs=[
            pl.BlockSpec((NCORES, RBLK, D), lambda i: (0, i, 0)),
            pl.BlockSpec((NCORES, RBLK, 16), lambda i: (0, i, 0)),
            pl.BlockSpec((16, D), lambda i: (0, 0)),
            pl.BlockSpec((1, D), lambda i: (0, 0)),
            pl.BlockSpec((1, D), lambda i: (0, 0)),
            pl.BlockSpec((1, D), lambda i: (0, 0)),
            pl.BlockSpec((D, D), lambda i: (0, 0)),
            pl.BlockSpec((D, 16), lambda i: (0, 0)),
            pl.BlockSpec((D, 16), lambda i: (0, 0)),
        ],
        out_specs=[
            pl.BlockSpec((RBLK, D), lambda i: (i, 0)),
            pl.BlockSpec((RBLK, 16), lambda i: (i, 0)),
            pl.BlockSpec((RBLK, 16), lambda i: (i, 0)),
        ],
        out_shape=[
            jax.ShapeDtypeStruct((N_PAD, D), _F32),
            jax.ShapeDtypeStruct((N_PAD, 16), _F32),
            jax.ShapeDtypeStruct((N_PAD, 16), _F32),
        ],
    )(num, den, rep, b1, bnw, bnb, w2, ms, md)


def _fin_body(num_ref, den_ref, rep_ref, b2_ref, o_ref):
    nm = num_ref[0] + num_ref[1]
    dn = den_ref[0] + den_ref[1]
    rep = _dot(dn, rep_ref[...])
    o_ref[...] = nm / (rep + 1e-16) + b2_ref[...]


def _tc_final(num, den, rep, b2):
    return pl.pallas_call(
        _fin_body,
        grid=(N_PAD // RBLK,),
        in_specs=[
            pl.BlockSpec((NCORES, RBLK, D), lambda i: (0, i, 0)),
            pl.BlockSpec((NCORES, RBLK, 16), lambda i: (0, i, 0)),
            pl.BlockSpec((16, D), lambda i: (0, 0)),
            pl.BlockSpec((1, D), lambda i: (0, 0)),
        ],
        out_specs=pl.BlockSpec((RBLK, D), lambda i: (i, 0)),
        out_shape=jax.ShapeDtypeStruct((N_PAD, D), _F32),
    )(num, den, rep, b2)


def _sc_edge_body(src2_hbm, dst2_hbm, stab_hbm, dtab_hbm, h_hbm,
                  z128_hbm, z16_hbm,
                  num_hbm, den_hbm,
                  sidx0, didx0, sidx1, didx1,
                  hrows0, hrows1, srows, drows, wbuf,
                  num_sh, den_sh, semi, semg0, semg1, semgs, semz, semsc0, semsc1, semds):
    cid = lax.axis_index("c")
    sid = lax.axis_index("s")
    wid = sid * NCORES + cid

    IDX = ((sidx0, didx0), (sidx1, didx1))
    HROWS = (hrows0, hrows1)
    SEMG = (semg0, semg1)
    SEMSC = (semsc0, semsc1)

    def issue_idx(k, bi):
        pltpu.async_copy(src2_hbm.at[wid * NCH + k], IDX[bi][0], semi)
        pltpu.async_copy(dst2_hbm.at[wid * NCH + k], IDX[bi][1], semi)

    def wait_idx(bi):
        pltpu.make_async_copy(src2_hbm.at[0], IDX[bi][0], semi).wait()
        pltpu.make_async_copy(dst2_hbm.at[0], IDX[bi][1], semi).wait()

    def issue_h(bi):
        pltpu.async_copy(h_hbm.at[IDX[bi][0]], HROWS[bi], SEMG[bi])

    def wait_h(bi):
        pltpu.make_async_copy(h_hbm.at[IDX[bi][0]], HROWS[bi], SEMG[bi]).wait()

    def issue_sd(bi):
        pltpu.async_copy(stab_hbm.at[IDX[bi][0]], srows, semgs)
        pltpu.async_copy(dtab_hbm.at[IDX[bi][1]], drows, semgs)

    def wait_sd(bi):
        pltpu.make_async_copy(stab_hbm.at[IDX[bi][0]], srows, semgs).wait()
        pltpu.make_async_copy(dtab_hbm.at[IDX[bi][1]], drows, semgs).wait()

    r0 = sid * ROWS_PER_SUB
    pltpu.async_copy(z128_hbm.at[pl.ds(r0, ROWS_PER_SUB)],
                     num_sh.at[pl.ds(r0, ROWS_PER_SUB)], semz)
    pltpu.async_copy(z16_hbm.at[pl.ds(r0, ROWS_PER_SUB)],
                     den_sh.at[pl.ds(r0, ROWS_PER_SUB)], semz)
    issue_idx(0, 0)
    wait_idx(0)
    issue_h(0)
    issue_sd(0)
    issue_idx(1, 1)
    pltpu.make_async_copy(z128_hbm.at[pl.ds(r0, ROWS_PER_SUB)],
                          num_sh.at[pl.ds(r0, ROWS_PER_SUB)], semz).wait()
    pltpu.make_async_copy(z16_hbm.at[pl.ds(r0, ROWS_PER_SUB)],
                          den_sh.at[pl.ds(r0, ROWS_PER_SUB)], semz).wait()
    plsc.subcore_barrier()

    def outer(j, carry):
        for b in range(2):
            k = 2 * j + b
            nb = 1 - b
            hrows = HROWS[b]
            didx = IDX[b][1]

            @pl.when(k < NCH - 1)
            def _():
                wait_idx(nb)

                @pl.when(k >= 1)
                def _():
                    pltpu.make_async_copy(
                        HROWS[nb], num_sh.at[IDX[nb][1]], SEMSC[nb]).wait()

                issue_h(nb)

            wait_sd(b)

            @pl.when(k >= 1)
            def _():
                pltpu.make_async_copy(wbuf, den_sh.at[didx], semds).wait()

            @plsc.parallel_loop(0, CHUNK)
            def w_body(e):
                t = srows[e, :] + drows[e, :]
                t = jnp.where(t > 0.0, t, 0.2 * t)
                wbuf[e, :] = jnp.exp(t)
            wait_h(b)

            @plsc.parallel_loop(0, CHUNK, unroll=2)
            def m_body(e):
                wrow = wbuf[e, :]
                for hd in range(H):
                    hrows[e, pl.ds(hd * C1, C1)] = (
                        hrows[e, pl.ds(hd * C1, C1)] * wrow[hd])
            pltpu.async_copy(hrows, num_sh.at[didx], SEMSC[b], add=True)
            pltpu.async_copy(wbuf, den_sh.at[didx], semds, add=True)

            @pl.when(k < NCH - 2)
            def _():
                issue_idx(k + 2, b)

            @pl.when(k < NCH - 1)
            def _():
                issue_sd(nb)
        return carry

    lax.fori_loop(0, NCH // 2, outer, 0)
    pltpu.make_async_copy(HROWS[0], num_sh.at[IDX[0][1]], SEMSC[0]).wait()
    pltpu.make_async_copy(HROWS[1], num_sh.at[IDX[1][1]], SEMSC[1]).wait()
    pltpu.make_async_copy(wbuf, den_sh.at[IDX[1][1]], semds).wait()
    plsc.subcore_barrier()

    pltpu.sync_copy(num_sh.at[pl.ds(r0, ROWS_PER_SUB)],
                    num_hbm.at[cid, pl.ds(r0, ROWS_PER_SUB)])
    pltpu.sync_copy(den_sh.at[pl.ds(r0, ROWS_PER_SUB)],
                    den_hbm.at[cid, pl.ds(r0, ROWS_PER_SUB)])


def _make_sc_edge():
  return pl.kernel(
    _sc_edge_body,
    out_type=[
        jax.ShapeDtypeStruct((NCORES, N_PAD, D), _F32),
        jax.ShapeDtypeStruct((NCORES, N_PAD, 16), _F32),
    ],
    mesh=plsc.VectorSubcoreMesh(core_axis_name="c", subcore_axis_name="s",
                                num_cores=NCORES, num_subcores=NSUB),
    compiler_params=pltpu.CompilerParams(use_tc_tiling_on_sc=False),
    scratch_types=[
        pltpu.VMEM((CHUNK,), jnp.int32),
        pltpu.VMEM((CHUNK,), jnp.int32),
        pltpu.VMEM((CHUNK,), jnp.int32),
        pltpu.VMEM((CHUNK,), jnp.int32),
        pltpu.VMEM((CHUNK, D), _F32),
        pltpu.VMEM((CHUNK, D), _F32),
        pltpu.VMEM((CHUNK, 16), _F32),
        pltpu.VMEM((CHUNK, 16), _F32),
        pltpu.VMEM((CHUNK, 16), _F32),
        pltpu.VMEM_SHARED((N_PAD, D), _F32),
        pltpu.VMEM_SHARED((N_PAD, 16), _F32),
        pltpu.SemaphoreType.DMA,
        pltpu.SemaphoreType.DMA,
        pltpu.SemaphoreType.DMA,
        pltpu.SemaphoreType.DMA,
        pltpu.SemaphoreType.DMA,
        pltpu.SemaphoreType.DMA,
        pltpu.SemaphoreType.DMA,
        pltpu.SemaphoreType.DMA,
    ],
  )


def kernel(x, edge_index, W1, a_src1, a_dst1, b1, bn_w, bn_b,
           W2, a_src2, a_dst2, b2):
    src = edge_index[0]
    dst = edge_index[1]

    eye = jnp.eye(H, dtype=_F32)
    a1s = a_src1.reshape(H, C1)
    a1d = a_dst1.reshape(H, C1)
    ms1h = (a1s[:, :, None] * eye[:, None, :]).reshape(H * C1, H)
    md1h = (a1d[:, :, None] * eye[:, None, :]).reshape(H * C1, H)
    ms1 = jnp.concatenate([ms1h, ms1h], axis=1)
    md1 = jnp.concatenate([md1h, md1h], axis=1)
    ms2 = jnp.tile(a_src2.reshape(D, 1), (1, 16))
    md2 = jnp.tile(a_dst2.reshape(D, 1), (1, 16))
    rep1 = jnp.concatenate(
        [jnp.kron(eye, jnp.ones((1, C1), _F32)), jnp.zeros((H, D), _F32)], axis=0)
    rep2 = jnp.full((16, D), 1.0 / 16.0, _F32)

    bnw_s = (bn_w / jnp.sqrt(1.0 + 1e-5)).reshape(1, D)
    z128 = jnp.zeros((N_PAD, D), _F32)
    z16 = jnp.zeros((N_PAD, 16), _F32)

    sc_edge = _make_sc_edge()
    pad = N + (jnp.arange(E_PAD - E, dtype=jnp.int32) % (N_PAD - N))
    src2 = jnp.concatenate([src, pad]).reshape(NCH * NW, CHUNK)
    dst2 = jnp.concatenate([dst, pad]).reshape(NCH * NW, CHUNK)
    xp = jnp.pad(x, ((0, N_PAD - N), (0, 0)))
    h1, s1, d1 = _tc_project(xp, W1, ms1, md1)
    num1, den1 = sc_edge(src2, dst2, s1, d1, h1, z128, z16)
    h2, s2, d2 = _tc_mid(num1, den1, rep1, b1.reshape(1, D), bnw_s,
                         bn_b.reshape(1, D), W2, ms2, md2)
    num2, den2 = sc_edge(src2, dst2, s2, d2, h2, z128, z16)
    return _tc_final(num2, den2, rep2, b2.reshape(1, D))[:N]


# speedup vs baseline: 1.7178x; 1.0862x over previous
import jax
import jax.numpy as jnp
from jax import lax
from jax.experimental import pallas as pl
from jax.experimental.pallas import tpu as pltpu
from jax.experimental.pallas import tpu_sc as plsc

N = 10000
N_PAD = 10112
E = 320000
D = 128
H = 8
C1 = 16
NCORES = 2
NSUB = 16
NW = NCORES * NSUB
CHUNK = 128
NCH = 80
E_PAD = NCH * CHUNK * NW
ROWS_PER_SUB = N_PAD // NSUB
RBLK = 1264

_HI = lax.Precision.HIGHEST
_F32 = jnp.float32


def _dot(a, b):
    return jnp.dot(a, b, precision=_HI, preferred_element_type=_F32)


def _proj_body(x_ref, w_ref, ms_ref, md_ref, h_ref, s_ref, d_ref):
    h = _dot(x_ref[...], w_ref[...])
    h_ref[...] = h
    s_ref[...] = _dot(h, ms_ref[...])
    d_ref[...] = _dot(h, md_ref[...])


def _tc_project(x, w, ms, md):
    return pl.pallas_call(
        _proj_body,
        grid=(N_PAD // RBLK,),
        in_specs=[
            pl.BlockSpec((RBLK, D), lambda i: (i, 0)),
            pl.BlockSpec((D, D), lambda i: (0, 0)),
            pl.BlockSpec((D, 16), lambda i: (0, 0)),
            pl.BlockSpec((D, 16), lambda i: (0, 0)),
        ],
        out_specs=[
            pl.BlockSpec((RBLK, D), lambda i: (i, 0)),
            pl.BlockSpec((RBLK, 16), lambda i: (i, 0)),
            pl.BlockSpec((RBLK, 16), lambda i: (i, 0)),
        ],
        out_shape=[
            jax.ShapeDtypeStruct((N_PAD, D), _F32),
            jax.ShapeDtypeStruct((N_PAD, 16), _F32),
            jax.ShapeDtypeStruct((N_PAD, 16), _F32),
        ],
    )(x, w, ms, md)


def _mid_body(num_ref, den_ref, rep_ref, b1_ref, bnw_ref, bnb_ref,
              w2_ref, ms_ref, md_ref, h_ref, s_ref, d_ref):
    nm = num_ref[0] + num_ref[1]
    dn = den_ref[0] + den_ref[1]
    rep = _dot(dn, rep_ref[...])
    g = nm / (rep + 1e-16) + b1_ref[...]
    t = g * bnw_ref[...] + bnb_ref[...]
    t = jnp.where(t > 0.0, t, jnp.exp(t) - 1.0)
    h2 = _dot(t, w2_ref[...])
    h_ref[...] = h2
    s_ref[...] = _dot(h2, ms_ref[...])
    d_ref[...] = _dot(h2, md_ref[...])


def _tc_mid(num, den, rep, b1, bnw, bnb, w2, ms, md):
    return pl.pallas_call(
        _mid_body,
        grid=(N_PAD // RBLK,),
        in_specs=[
            pl.BlockSpec((NCORES, RBLK, D), lambda i: (0, i, 0)),
            pl.BlockSpec((NCORES, RBLK, 16), lambda i: (0, i, 0)),
            pl.BlockSpec((16, D), lambda i: (0, 0)),
            pl.BlockSpec((1, D), lambda i: (0, 0)),
            pl.BlockSpec((1, D), lambda i: (0, 0)),
            pl.BlockSpec((1, D), lambda i: (0, 0)),
            pl.BlockSpec((D, D), lambda i: (0, 0)),
            pl.BlockSpec((D, 16), lambda i: (0, 0)),
            pl.BlockSpec((D, 16), lambda i: (0, 0)),
        ],
        out_specs=[
            pl.BlockSpec((RBLK, D), lambda i: (i, 0)),
            pl.BlockSpec((RBLK, 16), lambda i: (i, 0)),
            pl.BlockSpec((RBLK, 16), lambda i: (i, 0)),
        ],
        out_shape=[
            jax.ShapeDtypeStruct((N_PAD, D), _F32),
            jax.ShapeDtypeStruct((N_PAD, 16), _F32),
            jax.ShapeDtypeStruct((N_PAD, 16), _F32),
        ],
    )(num, den, rep, b1, bnw, bnb, w2, ms, md)


def _fin_body(num_ref, den_ref, rep_ref, b2_ref, o_ref):
    nm = num_ref[0] + num_ref[1]
    dn = den_ref[0] + den_ref[1]
    rep = _dot(dn, rep_ref[...])
    o_ref[...] = nm / (rep + 1e-16) + b2_ref[...]


def _tc_final(num, den, rep, b2):
    return pl.pallas_call(
        _fin_body,
        grid=(N_PAD // RBLK,),
        in_specs=[
            pl.BlockSpec((NCORES, RBLK, D), lambda i: (0, i, 0)),
            pl.BlockSpec((NCORES, RBLK, 16), lambda i: (0, i, 0)),
            pl.BlockSpec((16, D), lambda i: (0, 0)),
            pl.BlockSpec((1, D), lambda i: (0, 0)),
        ],
        out_specs=pl.BlockSpec((RBLK, D), lambda i: (i, 0)),
        out_shape=jax.ShapeDtypeStruct((N_PAD, D), _F32),
    )(num, den, rep, b2)


def _sc_edge_body(src2_hbm, dst2_hbm, stab_hbm, dtab_hbm, h_hbm,
                  z128_hbm, z16_hbm,
                  num_hbm, den_hbm,
                  sidx0, didx0, sidx1, didx1,
                  hrows0, hrows1, srows, drows, wbuf,
                  num_sh, den_sh, semi, semg0, semg1, semgs, semz, semsc0, semsc1, semds):
    cid = lax.axis_index("c")
    sid = lax.axis_index("s")
    wid = sid * NCORES + cid

    IDX = ((sidx0, didx0), (sidx1, didx1))
    HROWS = (hrows0, hrows1)
    SEMG = (semg0, semg1)
    SEMSC = (semsc0, semsc1)

    def issue_idx(k, bi):
        pltpu.async_copy(src2_hbm.at[wid * NCH + k], IDX[bi][0], semi)
        pltpu.async_copy(dst2_hbm.at[wid * NCH + k], IDX[bi][1], semi)

    def wait_idx(bi):
        pltpu.make_async_copy(src2_hbm.at[0], IDX[bi][0], semi).wait()
        pltpu.make_async_copy(dst2_hbm.at[0], IDX[bi][1], semi).wait()

    def issue_h(bi):
        pltpu.async_copy(h_hbm.at[IDX[bi][0]], HROWS[bi], SEMG[bi])

    def wait_h(bi):
        pltpu.make_async_copy(h_hbm.at[IDX[bi][0]], HROWS[bi], SEMG[bi]).wait()

    def issue_sd(bi):
        pltpu.async_copy(stab_hbm.at[IDX[bi][0]], srows, semgs)
        pltpu.async_copy(dtab_hbm.at[IDX[bi][1]], drows, semgs)

    def wait_sd(bi):
        pltpu.make_async_copy(stab_hbm.at[IDX[bi][0]], srows, semgs).wait()
        pltpu.make_async_copy(dtab_hbm.at[IDX[bi][1]], drows, semgs).wait()

    r0 = sid * ROWS_PER_SUB
    pltpu.async_copy(z128_hbm.at[pl.ds(r0, ROWS_PER_SUB)],
                     num_sh.at[pl.ds(r0, ROWS_PER_SUB)], semz)
    pltpu.async_copy(z16_hbm.at[pl.ds(r0, ROWS_PER_SUB)],
                     den_sh.at[pl.ds(r0, ROWS_PER_SUB)], semz)
    issue_idx(0, 0)
    wait_idx(0)
    issue_h(0)
    issue_sd(0)
    issue_idx(1, 1)
    pltpu.make_async_copy(z128_hbm.at[pl.ds(r0, ROWS_PER_SUB)],
                          num_sh.at[pl.ds(r0, ROWS_PER_SUB)], semz).wait()
    pltpu.make_async_copy(z16_hbm.at[pl.ds(r0, ROWS_PER_SUB)],
                          den_sh.at[pl.ds(r0, ROWS_PER_SUB)], semz).wait()
    plsc.subcore_barrier()

    def outer(j, carry):
        for b in range(2):
            k = 2 * j + b
            nb = 1 - b
            hrows = HROWS[b]
            didx = IDX[b][1]

            @pl.when(k < NCH - 1)
            def _():
                wait_idx(nb)

                @pl.when(k >= 1)
                def _():
                    pltpu.make_async_copy(
                        HROWS[nb], num_sh.at[IDX[nb][1]], SEMSC[nb]).wait()

                issue_h(nb)

            wait_sd(b)

            @pl.when(k >= 1)
            def _():
                pltpu.make_async_copy(wbuf, den_sh.at[didx], semds).wait()

            @plsc.parallel_loop(0, CHUNK, unroll=2)
            def w_body(e):
                t = srows[e, :] + drows[e, :]
                t = jnp.where(t > 0.0, t, 0.2 * t)
                wbuf[e, :] = jnp.exp(t)
            wait_h(b)

            @plsc.parallel_loop(0, CHUNK, unroll=4)
            def m_body(e):
                wrow = wbuf[e, :]
                for hd in range(H):
                    hrows[e, pl.ds(hd * C1, C1)] = (
                        hrows[e, pl.ds(hd * C1, C1)] * wrow[hd])
            pltpu.async_copy(hrows, num_sh.at[didx], SEMSC[b], add=True)
            pltpu.async_copy(wbuf, den_sh.at[didx], semds, add=True)

            @pl.when(k < NCH - 2)
            def _():
                issue_idx(k + 2, b)

            @pl.when(k < NCH - 1)
            def _():
                issue_sd(nb)
        return carry

    lax.fori_loop(0, NCH // 2, outer, 0)
    pltpu.make_async_copy(HROWS[0], num_sh.at[IDX[0][1]], SEMSC[0]).wait()
    pltpu.make_async_copy(HROWS[1], num_sh.at[IDX[1][1]], SEMSC[1]).wait()
    pltpu.make_async_copy(wbuf, den_sh.at[IDX[1][1]], semds).wait()
    plsc.subcore_barrier()

    pltpu.sync_copy(num_sh.at[pl.ds(r0, ROWS_PER_SUB)],
                    num_hbm.at[cid, pl.ds(r0, ROWS_PER_SUB)])
    pltpu.sync_copy(den_sh.at[pl.ds(r0, ROWS_PER_SUB)],
                    den_hbm.at[cid, pl.ds(r0, ROWS_PER_SUB)])


def _make_sc_edge():
  return pl.kernel(
    _sc_edge_body,
    out_type=[
        jax.ShapeDtypeStruct((NCORES, N_PAD, D), _F32),
        jax.ShapeDtypeStruct((NCORES, N_PAD, 16), _F32),
    ],
    mesh=plsc.VectorSubcoreMesh(core_axis_name="c", subcore_axis_name="s",
                                num_cores=NCORES, num_subcores=NSUB),
    compiler_params=pltpu.CompilerParams(use_tc_tiling_on_sc=False),
    scratch_types=[
        pltpu.VMEM((CHUNK,), jnp.int32),
        pltpu.VMEM((CHUNK,), jnp.int32),
        pltpu.VMEM((CHUNK,), jnp.int32),
        pltpu.VMEM((CHUNK,), jnp.int32),
        pltpu.VMEM((CHUNK, D), _F32),
        pltpu.VMEM((CHUNK, D), _F32),
        pltpu.VMEM((CHUNK, 16), _F32),
        pltpu.VMEM((CHUNK, 16), _F32),
        pltpu.VMEM((CHUNK, 16), _F32),
        pltpu.VMEM_SHARED((N_PAD, D), _F32),
        pltpu.VMEM_SHARED((N_PAD, 16), _F32),
        pltpu.SemaphoreType.DMA,
        pltpu.SemaphoreType.DMA,
        pltpu.SemaphoreType.DMA,
        pltpu.SemaphoreType.DMA,
        pltpu.SemaphoreType.DMA,
        pltpu.SemaphoreType.DMA,
        pltpu.SemaphoreType.DMA,
        pltpu.SemaphoreType.DMA,
    ],
  )


def kernel(x, edge_index, W1, a_src1, a_dst1, b1, bn_w, bn_b,
           W2, a_src2, a_dst2, b2):
    src = edge_index[0]
    dst = edge_index[1]

    eye = jnp.eye(H, dtype=_F32)
    a1s = a_src1.reshape(H, C1)
    a1d = a_dst1.reshape(H, C1)
    ms1h = (a1s[:, :, None] * eye[:, None, :]).reshape(H * C1, H)
    md1h = (a1d[:, :, None] * eye[:, None, :]).reshape(H * C1, H)
    ms1 = jnp.concatenate([ms1h, ms1h], axis=1)
    md1 = jnp.concatenate([md1h, md1h], axis=1)
    ms2 = jnp.tile(a_src2.reshape(D, 1), (1, 16))
    md2 = jnp.tile(a_dst2.reshape(D, 1), (1, 16))
    rep1 = jnp.concatenate(
        [jnp.kron(eye, jnp.ones((1, C1), _F32)), jnp.zeros((H, D), _F32)], axis=0)
    rep2 = jnp.full((16, D), 1.0 / 16.0, _F32)

    bnw_s = (bn_w / jnp.sqrt(1.0 + 1e-5)).reshape(1, D)
    z128 = jnp.zeros((N_PAD, D), _F32)
    z16 = jnp.zeros((N_PAD, 16), _F32)

    sc_edge = _make_sc_edge()
    pad = N + (jnp.arange(E_PAD - E, dtype=jnp.int32) % (N_PAD - N))
    src2 = jnp.concatenate([src, pad]).reshape(NCH * NW, CHUNK)
    dst2 = jnp.concatenate([dst, pad]).reshape(NCH * NW, CHUNK)
    xp = jnp.pad(x, ((0, N_PAD - N), (0, 0)))
    h1, s1, d1 = _tc_project(xp, W1, ms1, md1)
    num1, den1 = sc_edge(src2, dst2, s1, d1, h1, z128, z16)
    h2, s2, d2 = _tc_mid(num1, den1, rep1, b1.reshape(1, D), bnw_s,
                         bn_b.reshape(1, D), W2, ms2, md2)
    num2, den2 = sc_edge(src2, dst2, s2, d2, h2, z128, z16)
    return _tc_final(num2, den2, rep2, b2.reshape(1, D))[:N]
